# baseline (speedup 1.0000x reference)
import functools

import numpy as np
import jax
import jax.numpy as jnp
from jax import lax
from jax.experimental import pallas as pl
from jax.experimental.pallas import tpu as pltpu
from jax.experimental.pallas import tpu_sc as plsc

NC = 2
NS = 16
L = 16
R = 5
A = 3
CH = 16


S = 4


@functools.lru_cache(maxsize=None)
def _pe_tables_np(length: int, dim: int):
    pos = np.arange(0, length, S, dtype=np.float64)[:, None]
    k = np.arange(dim, dtype=np.float64)[None, :]
    rates = 1.0 / np.power(10000.0, (2.0 * np.floor(k / 2.0)) / dim)
    angles = pos * rates
    even = (np.arange(dim) % 2) == 0
    seed = np.where(even, np.sin(angles), np.cos(angles))
    seedsw = np.where(even, np.cos(angles), np.sin(angles))
    w = rates[0]
    rot = np.stack([np.cos(w), np.where(even, np.sin(w), -np.sin(w))])
    return (seed.astype(np.float32), seedsw.astype(np.float32),
            rot.astype(np.float32))


def kernel(x, table):
    B, T = x.shape
    V, D = table.shape
    NW = NC * NS
    POS_W = T // NW
    NCH = POS_W // CH
    assert T % NW == 0 and POS_W % CH == 0 and D % L == 0

    seed_np, seedsw_np, rot_np = _pe_tables_np(T, D)
    seed = jnp.asarray(seed_np)
    seedsw = jnp.asarray(seedsw_np)
    rot = jnp.asarray(rot_np)
    xf = x.reshape(B * T).astype(jnp.int32)
    Q = CH // S

    mesh = plsc.VectorSubcoreMesh(
        core_axis_name="c", subcore_axis_name="s",
        num_cores=NC, num_subcores=NS)

    @functools.partial(
        pl.kernel,
        out_type=jax.ShapeDtypeStruct((B * T, D), jnp.float32),
        mesh=mesh,
        scratch_types=(
            [pltpu.VMEM((B * POS_W,), jnp.int32)]
            + [pltpu.VMEM((CH, D), jnp.float32)]
            + [pltpu.VMEM((2, Q, D), jnp.float32)] * 2
            + [pltpu.VMEM((2, D), jnp.float32)]
            + [pltpu.VMEM((CH, D), jnp.float32)] * R
            + [pltpu.SemaphoreType.DMA] * (3 + 2 * R)
        ),
    )
    def sc_embed(x_hbm, seed_hbm, seedsw_hbm, rot_hbm, table_hbm, out_hbm,
                 idx_v, pe_v, sd0, sd1, rot_v,
                 r0, r1, r2, r3, r4,
                 si, sq0, sq1, sg0, sg1, sg2, sg3, sg4,
                 ss0, ss1, ss2, ss3, ss4):
        seed_bufs, seed_sems = [sd0, sd1], [sq0, sq1]
        row_bufs = [r0, r1, r2, r3, r4]
        g_sems = [sg0, sg1, sg2, sg3, sg4]
        s_sems = [ss0, ss1, ss2, ss3, ss4]

        wid = lax.axis_index("s") * NC + lax.axis_index("c")
        pbase = wid * POS_W
        sbase = wid * (POS_W // S)

        prelude_descs = [
            pltpu.async_copy(rot_hbm, rot_v, si),
        ] + [
            pltpu.async_copy(x_hbm.at[pl.ds(b * T + pbase, POS_W)],
                             idx_v.at[pl.ds(b * POS_W, POS_W)], si)
            for b in range(B)]
        for dsc in prelude_descs:
            dsc.wait()

        items = [(c, b) for c in range(NCH) for b in range(B)]
        n_items = len(items)
        seed_desc, gather_desc, store_desc = {}, {}, {}

        def start_seed(c):
            buf, sem = seed_bufs[c % 2], seed_sems[c % 2]
            seed_desc[c] = (
                pltpu.async_copy(seed_hbm.at[pl.ds(sbase + c * Q, Q)],
                                 buf.at[0], sem),
                pltpu.async_copy(seedsw_hbm.at[pl.ds(sbase + c * Q, Q)],
                                 buf.at[1], sem))

        def start_gather(i):
            if i - R in store_desc:
                store_desc.pop(i - R).wait()
            c, b = items[i]
            idx_ref = idx_v.at[pl.ds(b * POS_W + c * CH, CH)]
            gather_desc[i] = pltpu.async_copy(
                table_hbm.at[idx_ref], row_bufs[i % R], g_sems[i % R])

        start_seed(0)
        for i in range(min(A, n_items)):
            start_gather(i)

        for i in range(n_items):
            c, b = items[i]
            if b == 0:
                if c + 1 < NCH:
                    start_seed(c + 1)
                d0, d1 = seed_desc.pop(c)
                d0.wait()
                d1.wait()
                sbuf = seed_bufs[c % 2]

                def gen(g, carry, sbuf=sbuf):
                    sl = pl.ds(g * L, L)
                    cw = rot_v[0, sl]
                    sws = rot_v[1, sl]
                    for q in range(Q):
                        v = sbuf[0, q, sl]
                        u = sbuf[1, q, sl]
                        pe_v[q * S, sl] = v
                        for r in range(1, S):
                            v, u = v * cw + u * sws, u * cw - v * sws
                            pe_v[q * S + r, sl] = v
                    return carry

                lax.fori_loop(0, D // L, gen, None)

            gather_desc.pop(i).wait()
            if i + A < n_items:
                start_gather(i + A)

            buf = row_bufs[i % R]

            def add_row(r, carry, buf=buf):
                for dd in range(D // L):
                    sl = pl.ds(dd * L, L)
                    plsc.addupdate(buf.at[r, sl], pe_v[r, sl])
                return carry

            lax.fori_loop(0, CH, add_row, None)
            store_desc[i] = pltpu.async_copy(
                buf, out_hbm.at[pl.ds(b * T + pbase + c * CH, CH)],
                s_sems[i % R])

        for j in sorted(store_desc):
            store_desc.pop(j).wait()

    out = sc_embed(xf, seed, seedsw, rot, table)
    return out.reshape(B, T, D)


# speedup vs baseline: 1.1588x; 1.1588x over previous
import functools

import numpy as np
import jax
import jax.numpy as jnp
from jax import lax
from jax.experimental import pallas as pl
from jax.experimental.pallas import tpu as pltpu
from jax.experimental.pallas import tpu_sc as plsc

NC = 2
NS = 16
L = 16
R = 5
A = 3
CH = 16
U = 8


@functools.lru_cache(maxsize=None)
def _pe_tables_np(length: int, dim: int):
    pos = np.arange(0, length, CH, dtype=np.float64)[:, None]
    k = np.arange(dim, dtype=np.float64)[None, :]
    rates = 1.0 / np.power(10000.0, (2.0 * np.floor(k / 2.0)) / dim)
    angles = pos * rates
    even = (np.arange(dim) % 2) == 0
    seed = np.where(even, np.sin(angles), np.cos(angles))
    seedsw = np.where(even, np.cos(angles), np.sin(angles))
    w = rates[0]
    rot = np.stack([np.cos(w), np.where(even, np.sin(w), -np.sin(w))])
    return (seed.astype(np.float32), seedsw.astype(np.float32),
            rot.astype(np.float32))


def kernel(x, table):
    B, T = x.shape
    V, D = table.shape
    NW = NC * NS
    POS_W = T // NW
    NCH = POS_W // CH
    assert T % NW == 0 and POS_W % CH == 0 and D % (L * U) == 0

    seed_np, seedsw_np, rot_np = _pe_tables_np(T, D)
    seed = jnp.asarray(seed_np)
    seedsw = jnp.asarray(seedsw_np)
    rot = jnp.asarray(rot_np)
    xf = x.reshape(B * T).astype(jnp.int32)

    mesh = plsc.VectorSubcoreMesh(
        core_axis_name="c", subcore_axis_name="s",
        num_cores=NC, num_subcores=NS)

    @functools.partial(
        pl.kernel,
        out_type=jax.ShapeDtypeStruct((B * T, D), jnp.float32),
        mesh=mesh,
        scratch_types=(
            [pltpu.VMEM((B * POS_W,), jnp.int32)]
            + [pltpu.VMEM((CH, D), jnp.float32)]
            + [pltpu.VMEM((NCH, D), jnp.float32)]
            + [pltpu.VMEM((NCH, D), jnp.float32)]
            + [pltpu.VMEM((2, D), jnp.float32)]
            + [pltpu.VMEM((CH, D), jnp.float32)] * R
            + [pltpu.SemaphoreType.DMA] * (1 + 2 * R)
        ),
    )
    def sc_embed(x_hbm, seed_hbm, seedsw_hbm, rot_hbm, table_hbm, out_hbm,
                 idx_v, pe_v, seeds_v, seedsw_v, rot_v,
                 r0, r1, r2, r3, r4,
                 si, sg0, sg1, sg2, sg3, sg4,
                 ss0, ss1, ss2, ss3, ss4):
        row_bufs = [r0, r1, r2, r3, r4]
        g_sems = [sg0, sg1, sg2, sg3, sg4]
        s_sems = [ss0, ss1, ss2, ss3, ss4]

        wid = lax.axis_index("s") * NC + lax.axis_index("c")
        pbase = wid * POS_W
        sbase = wid * NCH

        prelude_descs = [
            pltpu.async_copy(rot_hbm, rot_v, si),
            pltpu.async_copy(seed_hbm.at[pl.ds(sbase, NCH)], seeds_v, si),
            pltpu.async_copy(seedsw_hbm.at[pl.ds(sbase, NCH)], seedsw_v, si),
        ] + [
            pltpu.async_copy(x_hbm.at[pl.ds(b * T + pbase, POS_W)],
                             idx_v.at[pl.ds(b * POS_W, POS_W)], si)
            for b in range(B)]
        for dsc in prelude_descs:
            dsc.wait()

        items = [(c, b) for c in range(NCH) for b in range(B)]
        n_items = len(items)
        gather_desc, store_desc = {}, {}

        def start_gather(i):
            if i - R in store_desc:
                store_desc.pop(i - R).wait()
            c, b = items[i]
            idx_ref = idx_v.at[pl.ds(b * POS_W + c * CH, CH)]
            gather_desc[i] = pltpu.async_copy(
                table_hbm.at[idx_ref], row_bufs[i % R], g_sems[i % R])

        for i in range(min(A, n_items)):
            start_gather(i)

        for i in range(n_items):
            c, b = items[i]
            if b == 0:
                def gen(g, carry, c=c):
                    sls = [pl.ds((g * U + j) * L, L) for j in range(U)]
                    cw = [rot_v[0, sl] for sl in sls]
                    sws = [rot_v[1, sl] for sl in sls]
                    v = [seeds_v[c, sl] for sl in sls]
                    u = [seedsw_v[c, sl] for sl in sls]
                    for j, sl in enumerate(sls):
                        pe_v[0, sl] = v[j]
                    for r in range(1, CH):
                        for j, sl in enumerate(sls):
                            v[j], u[j] = (v[j] * cw[j] + u[j] * sws[j],
                                          u[j] * cw[j] - v[j] * sws[j])
                            pe_v[r, sl] = v[j]
                    return carry

                lax.fori_loop(0, D // L // U, gen, None)

            gather_desc.pop(i).wait()
            if i + A < n_items:
                start_gather(i + A)

            buf = row_bufs[i % R]

            def add_row(r, carry, buf=buf):
                for dd in range(D // L):
                    sl = pl.ds(dd * L, L)
                    plsc.addupdate(buf.at[r, sl], pe_v[r, sl])
                return carry

            lax.fori_loop(0, CH, add_row, None)
            store_desc[i] = pltpu.async_copy(
                buf, out_hbm.at[pl.ds(b * T + pbase + c * CH, CH)],
                s_sems[i % R])

        for j in sorted(store_desc):
            store_desc.pop(j).wait()

    out = sc_embed(xf, seed, seedsw, rot, table)
    return out.reshape(B, T, D)


# speedup vs baseline: 1.2231x; 1.0554x over previous
import functools

import numpy as np
import jax
import jax.numpy as jnp
from jax import lax
from jax.experimental import pallas as pl
from jax.experimental.pallas import tpu as pltpu
from jax.experimental.pallas import tpu_sc as plsc

NC = 2
NS = 16
L = 16
R = 5
A = 3


@functools.lru_cache(maxsize=None)
def _pos_encoding_np(length: int, dim: int) -> np.ndarray:
    pos = np.arange(length, dtype=np.float64)[:, None]
    i = np.arange(dim, dtype=np.float64)[None, :]
    angle_rates = 1.0 / np.power(10000.0, (2.0 * np.floor(i / 2.0)) / dim)
    angles = pos * angle_rates
    pe = np.where((np.arange(dim) % 2) == 0, np.sin(angles), np.cos(angles))
    return pe.astype(np.float32)


def kernel(x, table):
    B, T = x.shape
    V, D = table.shape
    NW = NC * NS
    POS_W = T // NW
    CH = 16
    NCH = POS_W // CH
    assert T % NW == 0 and POS_W % CH == 0 and D % L == 0

    pe = jnp.asarray(_pos_encoding_np(T, D))
    xf = x.reshape(B * T).astype(jnp.int32)

    mesh = plsc.VectorSubcoreMesh(
        core_axis_name="c", subcore_axis_name="s",
        num_cores=NC, num_subcores=NS)

    @functools.partial(
        pl.kernel,
        out_type=jax.ShapeDtypeStruct((B * T, D), jnp.float32),
        mesh=mesh,
        scratch_types=(
            [pltpu.VMEM((B * POS_W,), jnp.int32)]
            + [pltpu.VMEM((CH, D), jnp.float32) for _ in range(2 + R)]
            + [pltpu.SemaphoreType.DMA for _ in range(3 + 2 * R)]
        ),
    )
    def sc_embed(x_hbm, pe_hbm, table_hbm, out_hbm, idx_v,
                 pe0, pe1, r0, r1, r2, r3, r4,
                 si, sp0, sp1, sg0, sg1, sg2, sg3, sg4,
                 ss0, ss1, ss2, ss3, ss4):
        pe_bufs, pe_sems = [pe0, pe1], [sp0, sp1]
        row_bufs = [r0, r1, r2, r3, r4]
        g_sems = [sg0, sg1, sg2, sg3, sg4]
        s_sems = [ss0, ss1, ss2, ss3, ss4]

        wid = lax.axis_index("s") * NC + lax.axis_index("c")
        pbase = wid * POS_W
        idx_descs = [
            pltpu.async_copy(x_hbm.at[pl.ds(b * T + pbase, POS_W)],
                             idx_v.at[pl.ds(b * POS_W, POS_W)], si)
            for b in range(B)]
        for dsc in idx_descs:
            dsc.wait()

        items = [(c, b) for c in range(NCH) for b in range(B)]
        n_items = len(items)
        pe_desc, gather_desc, store_desc = {}, {}, {}

        def start_pe(c):
            pe_desc[c] = pltpu.async_copy(
                pe_hbm.at[pl.ds(pbase + c * CH, CH)],
                pe_bufs[c % 2], pe_sems[c % 2])

        def start_gather(i):
            if i - R in store_desc:
                store_desc.pop(i - R).wait()
            c, b = items[i]
            idx_ref = idx_v.at[pl.ds(b * POS_W + c * CH, CH)]
            gather_desc[i] = pltpu.async_copy(
                table_hbm.at[idx_ref], row_bufs[i % R], g_sems[i % R])

        start_pe(0)
        for i in range(min(A, n_items)):
            start_gather(i)

        for i in range(n_items):
            c, b = items[i]
            if b == 0:
                if c + 1 < NCH:
                    start_pe(c + 1)
                pe_desc.pop(c).wait()
            gather_desc.pop(i).wait()
            if i + A < n_items:
                start_gather(i + A)

            buf, pe_buf = row_bufs[i % R], pe_bufs[c % 2]

            def add_row(r, carry, buf=buf, pe_buf=pe_buf):
                for dd in range(D // L):
                    sl = pl.ds(dd * L, L)
                    plsc.addupdate(buf.at[r, sl], pe_buf[r, sl])
                return carry

            lax.fori_loop(0, CH, add_row, None)
            store_desc[i] = pltpu.async_copy(
                buf, out_hbm.at[pl.ds(b * T + pbase + c * CH, CH)],
                s_sems[i % R])

        for j in sorted(store_desc):
            store_desc.pop(j).wait()

    out = sc_embed(xf, pe, table)
    return out.reshape(B, T, D)
